# Initial kernel scaffold; baseline (speedup 1.0000x reference)
import functools

import jax
import jax.numpy as jnp
from jax import lax
from jax.experimental import pallas as pl
from jax.experimental.pallas import tpu as pltpu
from jax.experimental.pallas import tpu_sc as plsc

N = 10000
S = 25
D = 256
H = 256
Z = 64

NC = 2
NS = 16
NW = NC * NS
GB = 4
RG = GB * S
NPW = 320
NG = NPW // GB
N_PAD = NW * NPW

BM = 2000
GRID = N // BM


def _gather_max(h_all, idx3):
    mesh = plsc.VectorSubcoreMesh(
        core_axis_name="c", subcore_axis_name="s",
        num_cores=NC, num_subcores=NS)

    @functools.partial(
        pl.kernel,
        out_type=jax.ShapeDtypeStruct((N_PAD, D), jnp.float32),
        mesh=mesh,
        scratch_types=[
            pltpu.VMEM((NG, RG), jnp.int32),
            pltpu.VMEM((2, RG, D), jnp.float32),
            pltpu.VMEM((GB, D), jnp.float32),
            pltpu.SemaphoreType.DMA,
            pltpu.SemaphoreType.DMA,
        ],
    )
    def k(h_hbm, idx_hbm, out_hbm, idx_v, rows_v, out_v, sem0, sem1):
        wid = lax.axis_index("s") * NC + lax.axis_index("c")
        node_base = wid * NPW
        pltpu.sync_copy(idx_hbm.at[wid], idx_v)
        sems = (sem0, sem1)
        pltpu.async_copy(h_hbm.at[idx_v.at[0]], rows_v.at[0], sem0)
        pltpu.async_copy(h_hbm.at[idx_v.at[1]], rows_v.at[1], sem1)

        def step(i, _):
            for hb in range(2):
                g = 2 * i + hb
                pltpu.make_async_copy(
                    h_hbm.at[idx_v.at[g]], rows_v.at[hb], sems[hb]).wait()

                def jloop(j, _):
                    col = j * 16
                    for b in range(GB):
                        r0 = b * S
                        acc = rows_v[hb, r0, pl.ds(col, 16)]
                        for s1 in range(1, S):
                            acc = jnp.maximum(
                                acc, rows_v[hb, r0 + s1, pl.ds(col, 16)])
                        out_v[b, pl.ds(col, 16)] = acc
                    return 0

                lax.fori_loop(0, D // 16, jloop, 0)
                pltpu.sync_copy(
                    out_v, out_hbm.at[pl.ds(node_base + g * GB, GB)])

                @pl.when(g + 2 < NG)
                def _():
                    pltpu.async_copy(
                        h_hbm.at[idx_v.at[g + 2]], rows_v.at[hb], sems[hb])
            return 0

        lax.fori_loop(0, NG // 2, step, 0)

    return k(h_all, idx3)


def _prep_idx(idx):
    flat = idx.reshape(-1).astype(jnp.int32)
    flat = jnp.pad(flat, (0, N_PAD * S - N * S))
    return flat.reshape(NW, NG, RG)


def _tc_stage1(x, w0t, b0, wet, be, g, beta):
    def body(x_ref, w0_ref, b0_ref, we_ref, be_ref, g_ref, bt_ref,
             h0_ref, ten_ref, st_ref, acc1, acc2):
        i = pl.program_id(0)
        xb = x_ref[...]
        h0 = jnp.dot(xb, w0_ref[...], preferred_element_type=jnp.float32)
        h0_ref[...] = jnp.maximum(h0 + b0_ref[...], 0.0)
        t = jnp.dot(xb, we_ref[...], preferred_element_type=jnp.float32)
        t = t + be_ref[...]
        ten_ref[...] = t

        @pl.when(i == 0)
        def _():
            acc1[...] = jnp.zeros_like(acc1)
            acc2[...] = jnp.zeros_like(acc2)

        acc1[...] += jnp.sum(t, axis=0, keepdims=True)
        acc2[...] += jnp.sum(t * t, axis=0, keepdims=True)

        @pl.when(i == GRID - 1)
        def _():
            mean = acc1[...] / N
            var = acc2[...] / N - mean * mean
            scale = g_ref[...] * lax.rsqrt(var + 1e-5)
            st_ref[0:1, :] = scale
            st_ref[1:2, :] = bt_ref[...] - mean * scale

    return pl.pallas_call(
        body,
        grid=(GRID,),
        in_specs=[
            pl.BlockSpec((BM, D), lambda i: (i, 0)),
            pl.BlockSpec((D, H), lambda i: (0, 0)),
            pl.BlockSpec((1, H), lambda i: (0, 0)),
            pl.BlockSpec((D, H), lambda i: (0, 0)),
            pl.BlockSpec((1, H), lambda i: (0, 0)),
            pl.BlockSpec((1, H), lambda i: (0, 0)),
            pl.BlockSpec((1, H), lambda i: (0, 0)),
        ],
        out_specs=[
            pl.BlockSpec((BM, H), lambda i: (i, 0)),
            pl.BlockSpec((BM, H), lambda i: (i, 0)),
            pl.BlockSpec((2, H), lambda i: (0, 0)),
        ],
        out_shape=[
            jax.ShapeDtypeStruct((N, H), jnp.float32),
            jax.ShapeDtypeStruct((N, H), jnp.float32),
            jax.ShapeDtypeStruct((2, H), jnp.float32),
        ],
        scratch_shapes=[
            pltpu.VMEM((1, H), jnp.float32),
            pltpu.VMEM((1, H), jnp.float32),
        ],
    )(x, w0t, b0, wet, be, g, beta)


def _tc_stage2(x, agg0, t_en, st, fc0at, fc0bt, fb0, w1t, b1, wezt, bez):
    def body(x_ref, a_ref, t_ref, st_ref, wa_ref, wb_ref, fb_ref,
             w1_ref, b1_ref, wz_ref, bz_ref, o_ref, h1_ref, z_ref):
        o = jnp.dot(x_ref[...], wa_ref[...], preferred_element_type=jnp.float32)
        o += jnp.dot(a_ref[...], wb_ref[...], preferred_element_type=jnp.float32)
        o = jnp.maximum(o + fb_ref[...], 0.0)
        o_ref[...] = o
        h1 = jnp.dot(o, w1_ref[...], preferred_element_type=jnp.float32)
        h1_ref[...] = jnp.maximum(h1 + b1_ref[...], 0.0)
        a = jnp.maximum(t_ref[...] * st_ref[0:1, :] + st_ref[1:2, :], 0.0)
        z = jnp.dot(a, wz_ref[...], preferred_element_type=jnp.float32)
        z_ref[...] = z + bz_ref[...]

    return pl.pallas_call(
        body,
        grid=(GRID,),
        in_specs=[
            pl.BlockSpec((BM, D), lambda i: (i, 0)),
            pl.BlockSpec((BM, D), lambda i: (i, 0)),
            pl.BlockSpec((BM, H), lambda i: (i, 0)),
            pl.BlockSpec((2, H), lambda i: (0, 0)),
            pl.BlockSpec((D, H), lambda i: (0, 0)),
            pl.BlockSpec((D, H), lambda i: (0, 0)),
            pl.BlockSpec((1, H), lambda i: (0, 0)),
            pl.BlockSpec((H, H), lambda i: (0, 0)),
            pl.BlockSpec((1, H), lambda i: (0, 0)),
            pl.BlockSpec((H, Z), lambda i: (0, 0)),
            pl.BlockSpec((1, Z), lambda i: (0, 0)),
        ],
        out_specs=[
            pl.BlockSpec((BM, H), lambda i: (i, 0)),
            pl.BlockSpec((BM, H), lambda i: (i, 0)),
            pl.BlockSpec((BM, Z), lambda i: (i, 0)),
        ],
        out_shape=[
            jax.ShapeDtypeStruct((N, H), jnp.float32),
            jax.ShapeDtypeStruct((N, H), jnp.float32),
            jax.ShapeDtypeStruct((N, Z), jnp.float32),
        ],
    )(x, agg0, t_en, st, fc0at, fc0bt, fb0, w1t, b1, wezt, bez)


def _tc_stage3(out1, agg1, ae_z, fc1at, fc1bt, fb1, wdht, bdh, g, beta):
    def body(o_ref, a_ref, ez_ref, wa_ref, wb_ref, fb_ref, wd_ref, bd_ref,
             g_ref, bt_ref, gs_ref, z_ref, td_ref, st_ref, acc1, acc2):
        i = pl.program_id(0)
        gs = jnp.dot(o_ref[...], wa_ref[...], preferred_element_type=jnp.float32)
        gs += jnp.dot(a_ref[...], wb_ref[...], preferred_element_type=jnp.float32)
        gs = gs + fb_ref[...]
        gs_ref[...] = gs
        z = 0.5 * ez_ref[...] + 0.5 * gs
        z_ref[...] = z
        t = jnp.dot(z, wd_ref[...], preferred_element_type=jnp.float32)
        t = t + bd_ref[...]
        td_ref[...] = t

        @pl.when(i == 0)
        def _():
            acc1[...] = jnp.zeros_like(acc1)
            acc2[...] = jnp.zeros_like(acc2)

        acc1[...] += jnp.sum(t, axis=0, keepdims=True)
        acc2[...] += jnp.sum(t * t, axis=0, keepdims=True)

        @pl.when(i == GRID - 1)
        def _():
            mean = acc1[...] / N
            var = acc2[...] / N - mean * mean
            scale = g_ref[...] * lax.rsqrt(var + 1e-5)
            st_ref[0:1, :] = scale
            st_ref[1:2, :] = bt_ref[...] - mean * scale

    return pl.pallas_call(
        body,
        grid=(GRID,),
        in_specs=[
            pl.BlockSpec((BM, H), lambda i: (i, 0)),
            pl.BlockSpec((BM, H), lambda i: (i, 0)),
            pl.BlockSpec((BM, Z), lambda i: (i, 0)),
            pl.BlockSpec((H, Z), lambda i: (0, 0)),
            pl.BlockSpec((H, Z), lambda i: (0, 0)),
            pl.BlockSpec((1, Z), lambda i: (0, 0)),
            pl.BlockSpec((Z, H), lambda i: (0, 0)),
            pl.BlockSpec((1, H), lambda i: (0, 0)),
            pl.BlockSpec((1, H), lambda i: (0, 0)),
            pl.BlockSpec((1, H), lambda i: (0, 0)),
        ],
        out_specs=[
            pl.BlockSpec((BM, Z), lambda i: (i, 0)),
            pl.BlockSpec((BM, Z), lambda i: (i, 0)),
            pl.BlockSpec((BM, H), lambda i: (i, 0)),
            pl.BlockSpec((2, H), lambda i: (0, 0)),
        ],
        out_shape=[
            jax.ShapeDtypeStruct((N, Z), jnp.float32),
            jax.ShapeDtypeStruct((N, Z), jnp.float32),
            jax.ShapeDtypeStruct((N, H), jnp.float32),
            jax.ShapeDtypeStruct((2, H), jnp.float32),
        ],
        scratch_shapes=[
            pltpu.VMEM((1, H), jnp.float32),
            pltpu.VMEM((1, H), jnp.float32),
        ],
    )(out1, agg1, ae_z, fc1at, fc1bt, fb1, wdht, bdh, g, beta)


def _tc_stage4(t_de, st, wdxt, bdx):
    def body(t_ref, st_ref, w_ref, b_ref, xb_ref):
        dd = jnp.maximum(t_ref[...] * st_ref[0:1, :] + st_ref[1:2, :], 0.0)
        xb = jnp.dot(dd, w_ref[...], preferred_element_type=jnp.float32)
        xb_ref[...] = xb + b_ref[...]

    return pl.pallas_call(
        body,
        grid=(GRID,),
        in_specs=[
            pl.BlockSpec((BM, H), lambda i: (i, 0)),
            pl.BlockSpec((2, H), lambda i: (0, 0)),
            pl.BlockSpec((H, D), lambda i: (0, 0)),
            pl.BlockSpec((1, D), lambda i: (0, 0)),
        ],
        out_specs=pl.BlockSpec((BM, D), lambda i: (i, 0)),
        out_shape=jax.ShapeDtypeStruct((N, D), jnp.float32),
    )(t_de, st, wdxt, bdx)


def kernel(x, neigh_idx0, neigh_idx1, agg0_W, agg0_b, agg1_W, agg1_b,
           fc0_W, fc0_b, fc1_W, fc1_b, en_h_W, en_h_b, en_g, en_beta,
           en_z_W, en_z_b, de_h_W, de_h_b, de_g, de_beta, de_x_W, de_x_b):
    r = lambda v: v.reshape(1, -1)
    idx3_0 = _prep_idx(neigh_idx0)
    idx3_1 = _prep_idx(neigh_idx1)

    h0_all, t_en, en_st = _tc_stage1(
        x, agg0_W.T, r(agg0_b), en_h_W.T, r(en_h_b), r(en_g), r(en_beta))
    agg0 = _gather_max(h0_all, idx3_0)[:N]
    out1, h1_all, ae_z = _tc_stage2(
        x, agg0, t_en, en_st, fc0_W[:, :D].T, fc0_W[:, D:].T, r(fc0_b),
        agg1_W.T, r(agg1_b), en_z_W.T, r(en_z_b))
    agg1 = _gather_max(h1_all, idx3_1)[:N]
    gs_z, z, t_de, de_st = _tc_stage3(
        out1, agg1, ae_z, fc1_W[:, :H].T, fc1_W[:, H:].T, r(fc1_b),
        de_h_W.T, r(de_h_b), r(de_g), r(de_beta))
    x_bar = _tc_stage4(t_de, de_st, de_x_W.T, r(de_x_b))
    return (ae_z, gs_z, z, x_bar, x)


# speedup vs baseline: 2.2202x; 2.2202x over previous
import functools

import jax
import jax.numpy as jnp
from jax import lax
from jax.experimental import pallas as pl
from jax.experimental.pallas import tpu as pltpu
from jax.experimental.pallas import tpu_sc as plsc

N = 10000
S = 25
D = 256
H = 256
Z = 64

NC = 2
NS = 16
NW = NC * NS
S2 = S + 1
GB = 4
RG = GB * S2
NPW = 320
NG = NPW // GB
N_PAD = NW * NPW
IW = NG * RG

BM = 2000
GRID = N // BM


def _gather_max(h_all, idx_flat):
    mesh = plsc.VectorSubcoreMesh(
        core_axis_name="c", subcore_axis_name="s",
        num_cores=NC, num_subcores=NS)

    @functools.partial(
        pl.kernel,
        out_type=jax.ShapeDtypeStruct((N_PAD, D), jnp.float32),
        mesh=mesh,
        scratch_types=[
            pltpu.VMEM((IW,), jnp.int32),
            pltpu.VMEM((2, RG, D), jnp.float32),
            pltpu.VMEM((2 * GB, D), jnp.float32),
            pltpu.SemaphoreType.DMA,
            pltpu.SemaphoreType.DMA,
        ],
    )
    def k(h_hbm, idx_hbm, out_hbm, idx_v, rows_v, out_v, sem0, sem1):
        wid = lax.axis_index("s") * NC + lax.axis_index("c")
        node_base = wid * NPW
        pltpu.sync_copy(idx_hbm.at[pl.ds(wid * IW, IW)], idx_v)
        sems = (sem0, sem1)
        pltpu.async_copy(
            h_hbm.at[idx_v.at[pl.ds(0, RG)]], rows_v.at[0], sem0)
        pltpu.async_copy(
            h_hbm.at[idx_v.at[pl.ds(RG, RG)]], rows_v.at[1], sem1)

        def step(i, _):
            for hb in range(2):
                g = 2 * i + hb
                pltpu.make_async_copy(
                    h_hbm.at[idx_v.at[pl.ds(g * RG, RG)]],
                    rows_v.at[hb], sems[hb]).wait()

                def jloop(j, _):
                    col = j * 16
                    for b in range(GB):
                        r0 = b * S2
                        acc = rows_v[hb, r0, pl.ds(col, 16)]
                        for s1 in range(1, S):
                            acc = jnp.maximum(
                                acc, rows_v[hb, r0 + s1, pl.ds(col, 16)])
                        out_v[hb * GB + b, pl.ds(col, 16)] = acc
                    return 0

                lax.fori_loop(0, D // 16, jloop, 0)

                @pl.when(g + 2 < NG)
                def _():
                    pltpu.async_copy(
                        h_hbm.at[idx_v.at[pl.ds((g + 2) * RG, RG)]],
                        rows_v.at[hb], sems[hb])
            pltpu.sync_copy(
                out_v, out_hbm.at[pl.ds(node_base + i * 2 * GB, 2 * GB)])
            return 0

        lax.fori_loop(0, NG // 2, step, 0)

    return k(h_all, idx_flat)


def _prep_idx(idx):
    idx = jnp.concatenate([idx, idx[:, :1]], axis=1).astype(jnp.int32)
    flat = idx.reshape(-1)
    return jnp.pad(flat, (0, N_PAD * S2 - N * S2))


def _tc_stage1(x, w0t, b0, wet, be, g, beta):
    def body(x_ref, w0_ref, b0_ref, we_ref, be_ref, g_ref, bt_ref,
             h0_ref, ten_ref, st_ref, acc1, acc2):
        i = pl.program_id(0)
        xb = x_ref[...]
        h0 = jnp.dot(xb, w0_ref[...], preferred_element_type=jnp.float32)
        h0_ref[...] = jnp.maximum(h0 + b0_ref[...], 0.0)
        t = jnp.dot(xb, we_ref[...], preferred_element_type=jnp.float32)
        t = t + be_ref[...]
        ten_ref[...] = t

        @pl.when(i == 0)
        def _():
            acc1[...] = jnp.zeros_like(acc1)
            acc2[...] = jnp.zeros_like(acc2)

        acc1[...] += jnp.sum(t, axis=0, keepdims=True)
        acc2[...] += jnp.sum(t * t, axis=0, keepdims=True)

        @pl.when(i == GRID - 1)
        def _():
            mean = acc1[...] / N
            var = acc2[...] / N - mean * mean
            scale = g_ref[...] * lax.rsqrt(var + 1e-5)
            st_ref[0:1, :] = scale
            st_ref[1:2, :] = bt_ref[...] - mean * scale

    return pl.pallas_call(
        body,
        grid=(GRID,),
        in_specs=[
            pl.BlockSpec((BM, D), lambda i: (i, 0)),
            pl.BlockSpec((D, H), lambda i: (0, 0)),
            pl.BlockSpec((1, H), lambda i: (0, 0)),
            pl.BlockSpec((D, H), lambda i: (0, 0)),
            pl.BlockSpec((1, H), lambda i: (0, 0)),
            pl.BlockSpec((1, H), lambda i: (0, 0)),
            pl.BlockSpec((1, H), lambda i: (0, 0)),
        ],
        out_specs=[
            pl.BlockSpec((BM, H), lambda i: (i, 0)),
            pl.BlockSpec((BM, H), lambda i: (i, 0)),
            pl.BlockSpec((2, H), lambda i: (0, 0)),
        ],
        out_shape=[
            jax.ShapeDtypeStruct((N, H), jnp.float32),
            jax.ShapeDtypeStruct((N, H), jnp.float32),
            jax.ShapeDtypeStruct((2, H), jnp.float32),
        ],
        scratch_shapes=[
            pltpu.VMEM((1, H), jnp.float32),
            pltpu.VMEM((1, H), jnp.float32),
        ],
    )(x, w0t, b0, wet, be, g, beta)


def _tc_stage2(x, agg0, t_en, st, fc0at, fc0bt, fb0, w1t, b1, wezt, bez):
    def body(x_ref, a_ref, t_ref, st_ref, wa_ref, wb_ref, fb_ref,
             w1_ref, b1_ref, wz_ref, bz_ref, o_ref, h1_ref, z_ref):
        o = jnp.dot(x_ref[...], wa_ref[...], preferred_element_type=jnp.float32)
        o += jnp.dot(a_ref[...], wb_ref[...], preferred_element_type=jnp.float32)
        o = jnp.maximum(o + fb_ref[...], 0.0)
        o_ref[...] = o
        h1 = jnp.dot(o, w1_ref[...], preferred_element_type=jnp.float32)
        h1_ref[...] = jnp.maximum(h1 + b1_ref[...], 0.0)
        a = jnp.maximum(t_ref[...] * st_ref[0:1, :] + st_ref[1:2, :], 0.0)
        z = jnp.dot(a, wz_ref[...], preferred_element_type=jnp.float32)
        z_ref[...] = z + bz_ref[...]

    return pl.pallas_call(
        body,
        grid=(GRID,),
        in_specs=[
            pl.BlockSpec((BM, D), lambda i: (i, 0)),
            pl.BlockSpec((BM, D), lambda i: (i, 0)),
            pl.BlockSpec((BM, H), lambda i: (i, 0)),
            pl.BlockSpec((2, H), lambda i: (0, 0)),
            pl.BlockSpec((D, H), lambda i: (0, 0)),
            pl.BlockSpec((D, H), lambda i: (0, 0)),
            pl.BlockSpec((1, H), lambda i: (0, 0)),
            pl.BlockSpec((H, H), lambda i: (0, 0)),
            pl.BlockSpec((1, H), lambda i: (0, 0)),
            pl.BlockSpec((H, Z), lambda i: (0, 0)),
            pl.BlockSpec((1, Z), lambda i: (0, 0)),
        ],
        out_specs=[
            pl.BlockSpec((BM, H), lambda i: (i, 0)),
            pl.BlockSpec((BM, H), lambda i: (i, 0)),
            pl.BlockSpec((BM, Z), lambda i: (i, 0)),
        ],
        out_shape=[
            jax.ShapeDtypeStruct((N, H), jnp.float32),
            jax.ShapeDtypeStruct((N, H), jnp.float32),
            jax.ShapeDtypeStruct((N, Z), jnp.float32),
        ],
    )(x, agg0, t_en, st, fc0at, fc0bt, fb0, w1t, b1, wezt, bez)


def _tc_stage3(out1, agg1, ae_z, fc1at, fc1bt, fb1, wdht, bdh, g, beta):
    def body(o_ref, a_ref, ez_ref, wa_ref, wb_ref, fb_ref, wd_ref, bd_ref,
             g_ref, bt_ref, gs_ref, z_ref, td_ref, st_ref, acc1, acc2):
        i = pl.program_id(0)
        gs = jnp.dot(o_ref[...], wa_ref[...], preferred_element_type=jnp.float32)
        gs += jnp.dot(a_ref[...], wb_ref[...], preferred_element_type=jnp.float32)
        gs = gs + fb_ref[...]
        gs_ref[...] = gs
        z = 0.5 * ez_ref[...] + 0.5 * gs
        z_ref[...] = z
        t = jnp.dot(z, wd_ref[...], preferred_element_type=jnp.float32)
        t = t + bd_ref[...]
        td_ref[...] = t

        @pl.when(i == 0)
        def _():
            acc1[...] = jnp.zeros_like(acc1)
            acc2[...] = jnp.zeros_like(acc2)

        acc1[...] += jnp.sum(t, axis=0, keepdims=True)
        acc2[...] += jnp.sum(t * t, axis=0, keepdims=True)

        @pl.when(i == GRID - 1)
        def _():
            mean = acc1[...] / N
            var = acc2[...] / N - mean * mean
            scale = g_ref[...] * lax.rsqrt(var + 1e-5)
            st_ref[0:1, :] = scale
            st_ref[1:2, :] = bt_ref[...] - mean * scale

    return pl.pallas_call(
        body,
        grid=(GRID,),
        in_specs=[
            pl.BlockSpec((BM, H), lambda i: (i, 0)),
            pl.BlockSpec((BM, H), lambda i: (i, 0)),
            pl.BlockSpec((BM, Z), lambda i: (i, 0)),
            pl.BlockSpec((H, Z), lambda i: (0, 0)),
            pl.BlockSpec((H, Z), lambda i: (0, 0)),
            pl.BlockSpec((1, Z), lambda i: (0, 0)),
            pl.BlockSpec((Z, H), lambda i: (0, 0)),
            pl.BlockSpec((1, H), lambda i: (0, 0)),
            pl.BlockSpec((1, H), lambda i: (0, 0)),
            pl.BlockSpec((1, H), lambda i: (0, 0)),
        ],
        out_specs=[
            pl.BlockSpec((BM, Z), lambda i: (i, 0)),
            pl.BlockSpec((BM, Z), lambda i: (i, 0)),
            pl.BlockSpec((BM, H), lambda i: (i, 0)),
            pl.BlockSpec((2, H), lambda i: (0, 0)),
        ],
        out_shape=[
            jax.ShapeDtypeStruct((N, Z), jnp.float32),
            jax.ShapeDtypeStruct((N, Z), jnp.float32),
            jax.ShapeDtypeStruct((N, H), jnp.float32),
            jax.ShapeDtypeStruct((2, H), jnp.float32),
        ],
        scratch_shapes=[
            pltpu.VMEM((1, H), jnp.float32),
            pltpu.VMEM((1, H), jnp.float32),
        ],
    )(out1, agg1, ae_z, fc1at, fc1bt, fb1, wdht, bdh, g, beta)


def _tc_stage4(t_de, st, wdxt, bdx):
    def body(t_ref, st_ref, w_ref, b_ref, xb_ref):
        dd = jnp.maximum(t_ref[...] * st_ref[0:1, :] + st_ref[1:2, :], 0.0)
        xb = jnp.dot(dd, w_ref[...], preferred_element_type=jnp.float32)
        xb_ref[...] = xb + b_ref[...]

    return pl.pallas_call(
        body,
        grid=(GRID,),
        in_specs=[
            pl.BlockSpec((BM, H), lambda i: (i, 0)),
            pl.BlockSpec((2, H), lambda i: (0, 0)),
            pl.BlockSpec((H, D), lambda i: (0, 0)),
            pl.BlockSpec((1, D), lambda i: (0, 0)),
        ],
        out_specs=pl.BlockSpec((BM, D), lambda i: (i, 0)),
        out_shape=jax.ShapeDtypeStruct((N, D), jnp.float32),
    )(t_de, st, wdxt, bdx)


def kernel(x, neigh_idx0, neigh_idx1, agg0_W, agg0_b, agg1_W, agg1_b,
           fc0_W, fc0_b, fc1_W, fc1_b, en_h_W, en_h_b, en_g, en_beta,
           en_z_W, en_z_b, de_h_W, de_h_b, de_g, de_beta, de_x_W, de_x_b):
    r = lambda v: v.reshape(1, -1)
    idx3_0 = _prep_idx(neigh_idx0)
    idx3_1 = _prep_idx(neigh_idx1)

    h0_all, t_en, en_st = _tc_stage1(
        x, agg0_W.T, r(agg0_b), en_h_W.T, r(en_h_b), r(en_g), r(en_beta))
    agg0 = _gather_max(h0_all, idx3_0)[:N]
    out1, h1_all, ae_z = _tc_stage2(
        x, agg0, t_en, en_st, fc0_W[:, :D].T, fc0_W[:, D:].T, r(fc0_b),
        agg1_W.T, r(agg1_b), en_z_W.T, r(en_z_b))
    agg1 = _gather_max(h1_all, idx3_1)[:N]
    gs_z, z, t_de, de_st = _tc_stage3(
        out1, agg1, ae_z, fc1_W[:, :H].T, fc1_W[:, H:].T, r(fc1_b),
        de_h_W.T, r(de_h_b), r(de_g), r(de_beta))
    x_bar = _tc_stage4(t_de, de_st, de_x_W.T, r(de_x_b))
    return (ae_z, gs_z, z, x_bar, x)


# speedup vs baseline: 2.2387x; 1.0083x over previous
import functools

import jax
import jax.numpy as jnp
from jax import lax
from jax.experimental import pallas as pl
from jax.experimental.pallas import tpu as pltpu
from jax.experimental.pallas import tpu_sc as plsc

N = 10000
S = 25
D = 256
H = 256
Z = 64

NC = 2
NS = 16
NW = NC * NS
S2 = S + 1
GB = 4
RG = GB * S2
NPW = 320
NG = NPW // GB
N_PAD = NW * NPW
IW = NG * RG

BM = 2000
GRID = N // BM


def _gather_max(h_all, idx_flat):
    mesh = plsc.VectorSubcoreMesh(
        core_axis_name="c", subcore_axis_name="s",
        num_cores=NC, num_subcores=NS)

    NB = 4

    @functools.partial(
        pl.kernel,
        out_type=jax.ShapeDtypeStruct((N_PAD, D), jnp.float32),
        mesh=mesh,
        scratch_types=[
            pltpu.VMEM((IW,), jnp.int32),
            pltpu.VMEM((NB, RG, D), jnp.float32),
            pltpu.VMEM((NB * GB, D), jnp.float32),
            [pltpu.SemaphoreType.DMA] * NB,
        ],
    )
    def k(h_hbm, idx_hbm, out_hbm, idx_v, rows_v, out_v, sems):
        wid = lax.axis_index("s") * NC + lax.axis_index("c")
        node_base = wid * NPW
        pltpu.sync_copy(idx_hbm.at[pl.ds(wid * IW, IW)], idx_v)
        for hb in range(NB):
            pltpu.async_copy(
                h_hbm.at[idx_v.at[pl.ds(hb * RG, RG)]], rows_v.at[hb],
                sems[hb])

        def step(i, _):
            for hb in range(NB):
                g = NB * i + hb
                pltpu.make_async_copy(
                    h_hbm.at[idx_v.at[pl.ds(g * RG, RG)]],
                    rows_v.at[hb], sems[hb]).wait()

                def jloop(j, _):
                    col = j * 16
                    for b in range(GB):
                        r0 = b * S2
                        acc0 = rows_v[hb, r0, pl.ds(col, 16)]
                        acc1 = rows_v[hb, r0 + 1, pl.ds(col, 16)]
                        for s1 in range(2, S, 2):
                            acc0 = jnp.maximum(
                                acc0, rows_v[hb, r0 + s1, pl.ds(col, 16)])
                        for s1 in range(3, S, 2):
                            acc1 = jnp.maximum(
                                acc1, rows_v[hb, r0 + s1, pl.ds(col, 16)])
                        out_v[hb * GB + b, pl.ds(col, 16)] = (
                            jnp.maximum(acc0, acc1))
                    return 0

                lax.fori_loop(0, D // 16, jloop, 0)

                @pl.when(g + NB < NG)
                def _():
                    pltpu.async_copy(
                        h_hbm.at[idx_v.at[pl.ds((g + NB) * RG, RG)]],
                        rows_v.at[hb], sems[hb])
            pltpu.sync_copy(
                out_v, out_hbm.at[pl.ds(node_base + i * NB * GB, NB * GB)])
            return 0

        lax.fori_loop(0, NG // NB, step, 0)

    return k(h_all, idx_flat)


def _prep_idx(idx):
    idx = jnp.concatenate([idx, idx[:, :1]], axis=1).astype(jnp.int32)
    flat = idx.reshape(-1)
    return jnp.pad(flat, (0, N_PAD * S2 - N * S2))


def _tc_stage1(x, w0t, b0, wet, be, g, beta):
    def body(x_ref, w0_ref, b0_ref, we_ref, be_ref, g_ref, bt_ref,
             h0_ref, ten_ref, st_ref, acc1, acc2):
        i = pl.program_id(0)
        xb = x_ref[...]
        h0 = jnp.dot(xb, w0_ref[...], preferred_element_type=jnp.float32)
        h0_ref[...] = jnp.maximum(h0 + b0_ref[...], 0.0)
        t = jnp.dot(xb, we_ref[...], preferred_element_type=jnp.float32)
        t = t + be_ref[...]
        ten_ref[...] = t

        @pl.when(i == 0)
        def _():
            acc1[...] = jnp.zeros_like(acc1)
            acc2[...] = jnp.zeros_like(acc2)

        acc1[...] += jnp.sum(t, axis=0, keepdims=True)
        acc2[...] += jnp.sum(t * t, axis=0, keepdims=True)

        @pl.when(i == GRID - 1)
        def _():
            mean = acc1[...] / N
            var = acc2[...] / N - mean * mean
            scale = g_ref[...] * lax.rsqrt(var + 1e-5)
            st_ref[0:1, :] = scale
            st_ref[1:2, :] = bt_ref[...] - mean * scale

    return pl.pallas_call(
        body,
        grid=(GRID,),
        in_specs=[
            pl.BlockSpec((BM, D), lambda i: (i, 0)),
            pl.BlockSpec((D, H), lambda i: (0, 0)),
            pl.BlockSpec((1, H), lambda i: (0, 0)),
            pl.BlockSpec((D, H), lambda i: (0, 0)),
            pl.BlockSpec((1, H), lambda i: (0, 0)),
            pl.BlockSpec((1, H), lambda i: (0, 0)),
            pl.BlockSpec((1, H), lambda i: (0, 0)),
        ],
        out_specs=[
            pl.BlockSpec((BM, H), lambda i: (i, 0)),
            pl.BlockSpec((BM, H), lambda i: (i, 0)),
            pl.BlockSpec((2, H), lambda i: (0, 0)),
        ],
        out_shape=[
            jax.ShapeDtypeStruct((N, H), jnp.float32),
            jax.ShapeDtypeStruct((N, H), jnp.float32),
            jax.ShapeDtypeStruct((2, H), jnp.float32),
        ],
        scratch_shapes=[
            pltpu.VMEM((1, H), jnp.float32),
            pltpu.VMEM((1, H), jnp.float32),
        ],
    )(x, w0t, b0, wet, be, g, beta)


def _tc_stage2(x, agg0, t_en, st, fc0at, fc0bt, fb0, w1t, b1, wezt, bez):
    def body(x_ref, a_ref, t_ref, st_ref, wa_ref, wb_ref, fb_ref,
             w1_ref, b1_ref, wz_ref, bz_ref, o_ref, h1_ref, z_ref):
        o = jnp.dot(x_ref[...], wa_ref[...], preferred_element_type=jnp.float32)
        o += jnp.dot(a_ref[...], wb_ref[...], preferred_element_type=jnp.float32)
        o = jnp.maximum(o + fb_ref[...], 0.0)
        o_ref[...] = o
        h1 = jnp.dot(o, w1_ref[...], preferred_element_type=jnp.float32)
        h1_ref[...] = jnp.maximum(h1 + b1_ref[...], 0.0)
        a = jnp.maximum(t_ref[...] * st_ref[0:1, :] + st_ref[1:2, :], 0.0)
        z = jnp.dot(a, wz_ref[...], preferred_element_type=jnp.float32)
        z_ref[...] = z + bz_ref[...]

    return pl.pallas_call(
        body,
        grid=(GRID,),
        in_specs=[
            pl.BlockSpec((BM, D), lambda i: (i, 0)),
            pl.BlockSpec((BM, D), lambda i: (i, 0)),
            pl.BlockSpec((BM, H), lambda i: (i, 0)),
            pl.BlockSpec((2, H), lambda i: (0, 0)),
            pl.BlockSpec((D, H), lambda i: (0, 0)),
            pl.BlockSpec((D, H), lambda i: (0, 0)),
            pl.BlockSpec((1, H), lambda i: (0, 0)),
            pl.BlockSpec((H, H), lambda i: (0, 0)),
            pl.BlockSpec((1, H), lambda i: (0, 0)),
            pl.BlockSpec((H, Z), lambda i: (0, 0)),
            pl.BlockSpec((1, Z), lambda i: (0, 0)),
        ],
        out_specs=[
            pl.BlockSpec((BM, H), lambda i: (i, 0)),
            pl.BlockSpec((BM, H), lambda i: (i, 0)),
            pl.BlockSpec((BM, Z), lambda i: (i, 0)),
        ],
        out_shape=[
            jax.ShapeDtypeStruct((N, H), jnp.float32),
            jax.ShapeDtypeStruct((N, H), jnp.float32),
            jax.ShapeDtypeStruct((N, Z), jnp.float32),
        ],
    )(x, agg0, t_en, st, fc0at, fc0bt, fb0, w1t, b1, wezt, bez)


def _tc_stage3(out1, agg1, ae_z, fc1at, fc1bt, fb1, wdht, bdh, g, beta):
    def body(o_ref, a_ref, ez_ref, wa_ref, wb_ref, fb_ref, wd_ref, bd_ref,
             g_ref, bt_ref, gs_ref, z_ref, td_ref, st_ref, acc1, acc2):
        i = pl.program_id(0)
        gs = jnp.dot(o_ref[...], wa_ref[...], preferred_element_type=jnp.float32)
        gs += jnp.dot(a_ref[...], wb_ref[...], preferred_element_type=jnp.float32)
        gs = gs + fb_ref[...]
        gs_ref[...] = gs
        z = 0.5 * ez_ref[...] + 0.5 * gs
        z_ref[...] = z
        t = jnp.dot(z, wd_ref[...], preferred_element_type=jnp.float32)
        t = t + bd_ref[...]
        td_ref[...] = t

        @pl.when(i == 0)
        def _():
            acc1[...] = jnp.zeros_like(acc1)
            acc2[...] = jnp.zeros_like(acc2)

        acc1[...] += jnp.sum(t, axis=0, keepdims=True)
        acc2[...] += jnp.sum(t * t, axis=0, keepdims=True)

        @pl.when(i == GRID - 1)
        def _():
            mean = acc1[...] / N
            var = acc2[...] / N - mean * mean
            scale = g_ref[...] * lax.rsqrt(var + 1e-5)
            st_ref[0:1, :] = scale
            st_ref[1:2, :] = bt_ref[...] - mean * scale

    return pl.pallas_call(
        body,
        grid=(GRID,),
        in_specs=[
            pl.BlockSpec((BM, H), lambda i: (i, 0)),
            pl.BlockSpec((BM, H), lambda i: (i, 0)),
            pl.BlockSpec((BM, Z), lambda i: (i, 0)),
            pl.BlockSpec((H, Z), lambda i: (0, 0)),
            pl.BlockSpec((H, Z), lambda i: (0, 0)),
            pl.BlockSpec((1, Z), lambda i: (0, 0)),
            pl.BlockSpec((Z, H), lambda i: (0, 0)),
            pl.BlockSpec((1, H), lambda i: (0, 0)),
            pl.BlockSpec((1, H), lambda i: (0, 0)),
            pl.BlockSpec((1, H), lambda i: (0, 0)),
        ],
        out_specs=[
            pl.BlockSpec((BM, Z), lambda i: (i, 0)),
            pl.BlockSpec((BM, Z), lambda i: (i, 0)),
            pl.BlockSpec((BM, H), lambda i: (i, 0)),
            pl.BlockSpec((2, H), lambda i: (0, 0)),
        ],
        out_shape=[
            jax.ShapeDtypeStruct((N, Z), jnp.float32),
            jax.ShapeDtypeStruct((N, Z), jnp.float32),
            jax.ShapeDtypeStruct((N, H), jnp.float32),
            jax.ShapeDtypeStruct((2, H), jnp.float32),
        ],
        scratch_shapes=[
            pltpu.VMEM((1, H), jnp.float32),
            pltpu.VMEM((1, H), jnp.float32),
        ],
    )(out1, agg1, ae_z, fc1at, fc1bt, fb1, wdht, bdh, g, beta)


def _tc_stage4(t_de, st, wdxt, bdx):
    def body(t_ref, st_ref, w_ref, b_ref, xb_ref):
        dd = jnp.maximum(t_ref[...] * st_ref[0:1, :] + st_ref[1:2, :], 0.0)
        xb = jnp.dot(dd, w_ref[...], preferred_element_type=jnp.float32)
        xb_ref[...] = xb + b_ref[...]

    return pl.pallas_call(
        body,
        grid=(GRID,),
        in_specs=[
            pl.BlockSpec((BM, H), lambda i: (i, 0)),
            pl.BlockSpec((2, H), lambda i: (0, 0)),
            pl.BlockSpec((H, D), lambda i: (0, 0)),
            pl.BlockSpec((1, D), lambda i: (0, 0)),
        ],
        out_specs=pl.BlockSpec((BM, D), lambda i: (i, 0)),
        out_shape=jax.ShapeDtypeStruct((N, D), jnp.float32),
    )(t_de, st, wdxt, bdx)


def kernel(x, neigh_idx0, neigh_idx1, agg0_W, agg0_b, agg1_W, agg1_b,
           fc0_W, fc0_b, fc1_W, fc1_b, en_h_W, en_h_b, en_g, en_beta,
           en_z_W, en_z_b, de_h_W, de_h_b, de_g, de_beta, de_x_W, de_x_b):
    r = lambda v: v.reshape(1, -1)
    idx3_0 = _prep_idx(neigh_idx0)
    idx3_1 = _prep_idx(neigh_idx1)

    h0_all, t_en, en_st = _tc_stage1(
        x, agg0_W.T, r(agg0_b), en_h_W.T, r(en_h_b), r(en_g), r(en_beta))
    agg0 = _gather_max(h0_all, idx3_0)[:N]
    out1, h1_all, ae_z = _tc_stage2(
        x, agg0, t_en, en_st, fc0_W[:, :D].T, fc0_W[:, D:].T, r(fc0_b),
        agg1_W.T, r(agg1_b), en_z_W.T, r(en_z_b))
    agg1 = _gather_max(h1_all, idx3_1)[:N]
    gs_z, z, t_de, de_st = _tc_stage3(
        out1, agg1, ae_z, fc1_W[:, :H].T, fc1_W[:, H:].T, r(fc1_b),
        de_h_W.T, r(de_h_b), r(de_g), r(de_beta))
    x_bar = _tc_stage4(t_de, de_st, de_x_W.T, r(de_x_b))
    return (ae_z, gs_z, z, x_bar, x)


# speedup vs baseline: 6.4141x; 2.8651x over previous
import functools

import jax
import jax.numpy as jnp
from jax import lax
from jax.experimental import pallas as pl
from jax.experimental.pallas import tpu as pltpu
from jax.experimental.pallas import tpu_sc as plsc

N = 10000
S = 25
D = 256
H = 256
Z = 64

NC = 2
NS = 16
NW = NC * NS
S2 = S + 1
GB = 4
RG = GB * S2
NPW = 320
NG = NPW // GB
N_PAD = NW * NPW
IW = NG * RG

BM = 2000
GRID = N // BM


def _gather_max(h_all, idx_flat):
    mesh = plsc.VectorSubcoreMesh(
        core_axis_name="c", subcore_axis_name="s",
        num_cores=NC, num_subcores=NS)

    NB = 4

    @functools.partial(
        pl.kernel,
        out_type=jax.ShapeDtypeStruct((N_PAD, D), jnp.float32),
        mesh=mesh,
        scratch_types=[
            [pltpu.VMEM((RG,), jnp.int32)] * NB,
            [pltpu.VMEM((RG, D), jnp.float32)] * NB,
            pltpu.VMEM((NB * GB, D), jnp.float32),
            [pltpu.SemaphoreType.DMA] * NB,
            [pltpu.SemaphoreType.DMA] * NB,
        ],
    )
    def k(h_hbm, idx_hbm, out_hbm, idxg, rows, out_v, isems, rsems):
        wid = lax.axis_index("s") * NC + lax.axis_index("c")
        node_base = wid * NPW
        ibase = wid * IW
        for hb in range(NB):
            pltpu.async_copy(
                idx_hbm.at[pl.ds(ibase + hb * RG, RG)], idxg[hb], isems[hb])
        for hb in range(NB):
            pltpu.make_async_copy(
                idx_hbm.at[pl.ds(ibase + hb * RG, RG)], idxg[hb],
                isems[hb]).wait()
            pltpu.async_copy(h_hbm.at[idxg[hb]], rows[hb], rsems[hb])

        def step(i, _):
            for hb in range(NB):
                g = NB * i + hb
                pltpu.make_async_copy(
                    h_hbm.at[idxg[hb]], rows[hb], rsems[hb]).wait()
                @pl.when(g + NB < NG)
                def _():
                    pltpu.async_copy(
                        idx_hbm.at[pl.ds(ibase + (g + NB) * RG, RG)],
                        idxg[hb], isems[hb])

                def jloop(j, _):
                    col = j * 16
                    for b in range(GB):
                        r0 = b * S2
                        acc0 = rows[hb][r0, pl.ds(col, 16)]
                        acc1 = rows[hb][r0 + 1, pl.ds(col, 16)]
                        for s1 in range(2, S, 2):
                            acc0 = jnp.maximum(
                                acc0, rows[hb][r0 + s1, pl.ds(col, 16)])
                        for s1 in range(3, S, 2):
                            acc1 = jnp.maximum(
                                acc1, rows[hb][r0 + s1, pl.ds(col, 16)])
                        out_v[hb * GB + b, pl.ds(col, 16)] = (
                            jnp.maximum(acc0, acc1))
                    return 0

                lax.fori_loop(0, D // 16, jloop, 0)

                @pl.when(g + NB < NG)
                def _():
                    pltpu.make_async_copy(
                        idx_hbm.at[pl.ds(ibase + (g + NB) * RG, RG)],
                        idxg[hb], isems[hb]).wait()
                    pltpu.async_copy(h_hbm.at[idxg[hb]], rows[hb], rsems[hb])
            pltpu.sync_copy(
                out_v, out_hbm.at[pl.ds(node_base + i * NB * GB, NB * GB)])
            return 0

        lax.fori_loop(0, NG // NB, step, 0)

    return k(h_all, idx_flat)


def _prep_idx(idx):
    idx = jnp.concatenate([idx, idx[:, :1]], axis=1).astype(jnp.int32)
    flat = idx.reshape(-1)
    pad = N_PAD * S2 - N * S2
    filler = (jnp.arange(pad, dtype=jnp.int32) * 37) % N
    return jnp.concatenate([flat, filler])


def _tc_stage1(x, w0t, b0, wet, be, g, beta):
    def body(x_ref, w0_ref, b0_ref, we_ref, be_ref, g_ref, bt_ref,
             h0_ref, ten_ref, st_ref, acc1, acc2):
        i = pl.program_id(0)
        xb = x_ref[...]
        h0 = jnp.dot(xb, w0_ref[...], preferred_element_type=jnp.float32)
        h0_ref[...] = jnp.maximum(h0 + b0_ref[...], 0.0)
        t = jnp.dot(xb, we_ref[...], preferred_element_type=jnp.float32)
        t = t + be_ref[...]
        ten_ref[...] = t

        @pl.when(i == 0)
        def _():
            acc1[...] = jnp.zeros_like(acc1)
            acc2[...] = jnp.zeros_like(acc2)

        acc1[...] += jnp.sum(t, axis=0, keepdims=True)
        acc2[...] += jnp.sum(t * t, axis=0, keepdims=True)

        @pl.when(i == GRID - 1)
        def _():
            mean = acc1[...] / N
            var = acc2[...] / N - mean * mean
            scale = g_ref[...] * lax.rsqrt(var + 1e-5)
            st_ref[0:1, :] = scale
            st_ref[1:2, :] = bt_ref[...] - mean * scale

    return pl.pallas_call(
        body,
        grid=(GRID,),
        in_specs=[
            pl.BlockSpec((BM, D), lambda i: (i, 0)),
            pl.BlockSpec((D, H), lambda i: (0, 0)),
            pl.BlockSpec((1, H), lambda i: (0, 0)),
            pl.BlockSpec((D, H), lambda i: (0, 0)),
            pl.BlockSpec((1, H), lambda i: (0, 0)),
            pl.BlockSpec((1, H), lambda i: (0, 0)),
            pl.BlockSpec((1, H), lambda i: (0, 0)),
        ],
        out_specs=[
            pl.BlockSpec((BM, H), lambda i: (i, 0)),
            pl.BlockSpec((BM, H), lambda i: (i, 0)),
            pl.BlockSpec((2, H), lambda i: (0, 0)),
        ],
        out_shape=[
            jax.ShapeDtypeStruct((N, H), jnp.float32),
            jax.ShapeDtypeStruct((N, H), jnp.float32),
            jax.ShapeDtypeStruct((2, H), jnp.float32),
        ],
        scratch_shapes=[
            pltpu.VMEM((1, H), jnp.float32),
            pltpu.VMEM((1, H), jnp.float32),
        ],
    )(x, w0t, b0, wet, be, g, beta)


def _tc_stage2(x, agg0, t_en, st, fc0at, fc0bt, fb0, w1t, b1, wezt, bez):
    def body(x_ref, a_ref, t_ref, st_ref, wa_ref, wb_ref, fb_ref,
             w1_ref, b1_ref, wz_ref, bz_ref, o_ref, h1_ref, z_ref):
        o = jnp.dot(x_ref[...], wa_ref[...], preferred_element_type=jnp.float32)
        o += jnp.dot(a_ref[...], wb_ref[...], preferred_element_type=jnp.float32)
        o = jnp.maximum(o + fb_ref[...], 0.0)
        o_ref[...] = o
        h1 = jnp.dot(o, w1_ref[...], preferred_element_type=jnp.float32)
        h1_ref[...] = jnp.maximum(h1 + b1_ref[...], 0.0)
        a = jnp.maximum(t_ref[...] * st_ref[0:1, :] + st_ref[1:2, :], 0.0)
        z = jnp.dot(a, wz_ref[...], preferred_element_type=jnp.float32)
        z_ref[...] = z + bz_ref[...]

    return pl.pallas_call(
        body,
        grid=(GRID,),
        in_specs=[
            pl.BlockSpec((BM, D), lambda i: (i, 0)),
            pl.BlockSpec((BM, D), lambda i: (i, 0)),
            pl.BlockSpec((BM, H), lambda i: (i, 0)),
            pl.BlockSpec((2, H), lambda i: (0, 0)),
            pl.BlockSpec((D, H), lambda i: (0, 0)),
            pl.BlockSpec((D, H), lambda i: (0, 0)),
            pl.BlockSpec((1, H), lambda i: (0, 0)),
            pl.BlockSpec((H, H), lambda i: (0, 0)),
            pl.BlockSpec((1, H), lambda i: (0, 0)),
            pl.BlockSpec((H, Z), lambda i: (0, 0)),
            pl.BlockSpec((1, Z), lambda i: (0, 0)),
        ],
        out_specs=[
            pl.BlockSpec((BM, H), lambda i: (i, 0)),
            pl.BlockSpec((BM, H), lambda i: (i, 0)),
            pl.BlockSpec((BM, Z), lambda i: (i, 0)),
        ],
        out_shape=[
            jax.ShapeDtypeStruct((N, H), jnp.float32),
            jax.ShapeDtypeStruct((N, H), jnp.float32),
            jax.ShapeDtypeStruct((N, Z), jnp.float32),
        ],
    )(x, agg0, t_en, st, fc0at, fc0bt, fb0, w1t, b1, wezt, bez)


def _tc_stage3(out1, agg1, ae_z, fc1at, fc1bt, fb1, wdht, bdh, g, beta):
    def body(o_ref, a_ref, ez_ref, wa_ref, wb_ref, fb_ref, wd_ref, bd_ref,
             g_ref, bt_ref, gs_ref, z_ref, td_ref, st_ref, acc1, acc2):
        i = pl.program_id(0)
        gs = jnp.dot(o_ref[...], wa_ref[...], preferred_element_type=jnp.float32)
        gs += jnp.dot(a_ref[...], wb_ref[...], preferred_element_type=jnp.float32)
        gs = gs + fb_ref[...]
        gs_ref[...] = gs
        z = 0.5 * ez_ref[...] + 0.5 * gs
        z_ref[...] = z
        t = jnp.dot(z, wd_ref[...], preferred_element_type=jnp.float32)
        t = t + bd_ref[...]
        td_ref[...] = t

        @pl.when(i == 0)
        def _():
            acc1[...] = jnp.zeros_like(acc1)
            acc2[...] = jnp.zeros_like(acc2)

        acc1[...] += jnp.sum(t, axis=0, keepdims=True)
        acc2[...] += jnp.sum(t * t, axis=0, keepdims=True)

        @pl.when(i == GRID - 1)
        def _():
            mean = acc1[...] / N
            var = acc2[...] / N - mean * mean
            scale = g_ref[...] * lax.rsqrt(var + 1e-5)
            st_ref[0:1, :] = scale
            st_ref[1:2, :] = bt_ref[...] - mean * scale

    return pl.pallas_call(
        body,
        grid=(GRID,),
        in_specs=[
            pl.BlockSpec((BM, H), lambda i: (i, 0)),
            pl.BlockSpec((BM, H), lambda i: (i, 0)),
            pl.BlockSpec((BM, Z), lambda i: (i, 0)),
            pl.BlockSpec((H, Z), lambda i: (0, 0)),
            pl.BlockSpec((H, Z), lambda i: (0, 0)),
            pl.BlockSpec((1, Z), lambda i: (0, 0)),
            pl.BlockSpec((Z, H), lambda i: (0, 0)),
            pl.BlockSpec((1, H), lambda i: (0, 0)),
            pl.BlockSpec((1, H), lambda i: (0, 0)),
            pl.BlockSpec((1, H), lambda i: (0, 0)),
        ],
        out_specs=[
            pl.BlockSpec((BM, Z), lambda i: (i, 0)),
            pl.BlockSpec((BM, Z), lambda i: (i, 0)),
            pl.BlockSpec((BM, H), lambda i: (i, 0)),
            pl.BlockSpec((2, H), lambda i: (0, 0)),
        ],
        out_shape=[
            jax.ShapeDtypeStruct((N, Z), jnp.float32),
            jax.ShapeDtypeStruct((N, Z), jnp.float32),
            jax.ShapeDtypeStruct((N, H), jnp.float32),
            jax.ShapeDtypeStruct((2, H), jnp.float32),
        ],
        scratch_shapes=[
            pltpu.VMEM((1, H), jnp.float32),
            pltpu.VMEM((1, H), jnp.float32),
        ],
    )(out1, agg1, ae_z, fc1at, fc1bt, fb1, wdht, bdh, g, beta)


def _tc_stage4(t_de, st, wdxt, bdx):
    def body(t_ref, st_ref, w_ref, b_ref, xb_ref):
        dd = jnp.maximum(t_ref[...] * st_ref[0:1, :] + st_ref[1:2, :], 0.0)
        xb = jnp.dot(dd, w_ref[...], preferred_element_type=jnp.float32)
        xb_ref[...] = xb + b_ref[...]

    return pl.pallas_call(
        body,
        grid=(GRID,),
        in_specs=[
            pl.BlockSpec((BM, H), lambda i: (i, 0)),
            pl.BlockSpec((2, H), lambda i: (0, 0)),
            pl.BlockSpec((H, D), lambda i: (0, 0)),
            pl.BlockSpec((1, D), lambda i: (0, 0)),
        ],
        out_specs=pl.BlockSpec((BM, D), lambda i: (i, 0)),
        out_shape=jax.ShapeDtypeStruct((N, D), jnp.float32),
    )(t_de, st, wdxt, bdx)


def kernel(x, neigh_idx0, neigh_idx1, agg0_W, agg0_b, agg1_W, agg1_b,
           fc0_W, fc0_b, fc1_W, fc1_b, en_h_W, en_h_b, en_g, en_beta,
           en_z_W, en_z_b, de_h_W, de_h_b, de_g, de_beta, de_x_W, de_x_b):
    r = lambda v: v.reshape(1, -1)
    idx3_0 = _prep_idx(neigh_idx0)
    idx3_1 = _prep_idx(neigh_idx1)

    h0_all, t_en, en_st = _tc_stage1(
        x, agg0_W.T, r(agg0_b), en_h_W.T, r(en_h_b), r(en_g), r(en_beta))
    agg0 = _gather_max(h0_all, idx3_0)[:N]
    out1, h1_all, ae_z = _tc_stage2(
        x, agg0, t_en, en_st, fc0_W[:, :D].T, fc0_W[:, D:].T, r(fc0_b),
        agg1_W.T, r(agg1_b), en_z_W.T, r(en_z_b))
    agg1 = _gather_max(h1_all, idx3_1)[:N]
    gs_z, z, t_de, de_st = _tc_stage3(
        out1, agg1, ae_z, fc1_W[:, :H].T, fc1_W[:, H:].T, r(fc1_b),
        de_h_W.T, r(de_h_b), r(de_g), r(de_beta))
    x_bar = _tc_stage4(t_de, de_st, de_x_W.T, r(de_x_b))
    return (ae_z, gs_z, z, x_bar, x)


# speedup vs baseline: 6.7825x; 1.0574x over previous
import functools

import jax
import jax.numpy as jnp
from jax import lax
from jax.experimental import pallas as pl
from jax.experimental.pallas import tpu as pltpu
from jax.experimental.pallas import tpu_sc as plsc

N = 10000
S = 25
D = 256
H = 256
Z = 64

NC = 2
NS = 16
NW = NC * NS
S2 = S + 1
GB = 4
RG = GB * S2
NPW = 320
NG = NPW // GB
N_PAD = NW * NPW
IW = NG * RG

BM = 2000
GRID = N // BM


def _gather_max(h_all, idx_flat):
    mesh = plsc.VectorSubcoreMesh(
        core_axis_name="c", subcore_axis_name="s",
        num_cores=NC, num_subcores=NS)

    NB = 4

    @functools.partial(
        pl.kernel,
        out_type=jax.ShapeDtypeStruct((N_PAD, D), jnp.float32),
        mesh=mesh,
        scratch_types=[
            [pltpu.VMEM((RG,), jnp.int32)] * NB,
            [pltpu.VMEM((RG, D), jnp.float32)] * NB,
            pltpu.VMEM((NB * GB, D), jnp.float32),
            [pltpu.SemaphoreType.DMA] * NB,
            [pltpu.SemaphoreType.DMA] * NB,
        ],
    )
    def k(h_hbm, idx_hbm, out_hbm, idxg, rows, out_v, isems, rsems):
        wid = lax.axis_index("s") * NC + lax.axis_index("c")
        node_base = wid * NPW
        ibase = wid * IW
        for hb in range(NB):
            pltpu.async_copy(
                idx_hbm.at[pl.ds(ibase + hb * RG, RG)], idxg[hb], isems[hb])
        for hb in range(NB):
            pltpu.make_async_copy(
                idx_hbm.at[pl.ds(ibase + hb * RG, RG)], idxg[hb],
                isems[hb]).wait()
            pltpu.async_copy(h_hbm.at[idxg[hb]], rows[hb], rsems[hb])

        def step(i, _):
            for hb in range(NB):
                g = NB * i + hb
                pltpu.make_async_copy(
                    h_hbm.at[idxg[hb]], rows[hb], rsems[hb]).wait()
                @pl.when(g + NB < NG)
                def _():
                    pltpu.async_copy(
                        idx_hbm.at[pl.ds(ibase + (g + NB) * RG, RG)],
                        idxg[hb], isems[hb])

                def jloop(j, _):
                    col = j * 16
                    for b in range(GB):
                        r0 = b * S2
                        acc0 = rows[hb][r0, pl.ds(col, 16)]
                        acc1 = rows[hb][r0 + 1, pl.ds(col, 16)]
                        for s1 in range(2, S, 2):
                            acc0 = jnp.maximum(
                                acc0, rows[hb][r0 + s1, pl.ds(col, 16)])
                        for s1 in range(3, S, 2):
                            acc1 = jnp.maximum(
                                acc1, rows[hb][r0 + s1, pl.ds(col, 16)])
                        out_v[hb * GB + b, pl.ds(col, 16)] = (
                            jnp.maximum(acc0, acc1))
                    return 0

                lax.fori_loop(0, D // 16, jloop, 0)

                @pl.when(g + NB < NG)
                def _():
                    pltpu.make_async_copy(
                        idx_hbm.at[pl.ds(ibase + (g + NB) * RG, RG)],
                        idxg[hb], isems[hb]).wait()
                    pltpu.async_copy(h_hbm.at[idxg[hb]], rows[hb], rsems[hb])
            pltpu.sync_copy(
                out_v, out_hbm.at[pl.ds(node_base + i * NB * GB, NB * GB)])
            return 0

        lax.fori_loop(0, NG // NB, step, 0)

    return k(h_all, idx_flat)


def _prep_idx(idx):
    idx = jnp.concatenate([idx, idx[:, :1]], axis=1).astype(jnp.int32)
    flat = idx.reshape(-1)
    pad = N_PAD * S2 - N * S2
    filler = (jnp.arange(pad, dtype=jnp.int32) * 37) % N
    return jnp.concatenate([flat, filler])


def _tc_stage1(x, w0t, b0):
    def body(x_ref, w0_ref, b0_ref, h0_ref):
        h0 = jnp.dot(x_ref[...], w0_ref[...],
                     preferred_element_type=jnp.float32)
        h0_ref[...] = jnp.maximum(h0 + b0_ref[...], 0.0)

    return pl.pallas_call(
        body,
        grid=(GRID,),
        in_specs=[
            pl.BlockSpec((BM, D), lambda i: (i, 0)),
            pl.BlockSpec((D, H), lambda i: (0, 0)),
            pl.BlockSpec((1, H), lambda i: (0, 0)),
        ],
        out_specs=pl.BlockSpec((BM, H), lambda i: (i, 0)),
        out_shape=jax.ShapeDtypeStruct((N, H), jnp.float32),
    )(x, w0t, b0)


def _tc_ae(x, wet, be, g, beta, wezt, bez):
    def body(x_ref, we_ref, be_ref, g_ref, bt_ref,
             ten_ref, st, acc1, acc2):
        i = pl.program_id(0)
        t = jnp.dot(x_ref[...], we_ref[...],
                    preferred_element_type=jnp.float32)
        t = t + be_ref[...]
        ten_ref[...] = t

        @pl.when(i == 0)
        def _():
            acc1[...] = jnp.zeros_like(acc1)
            acc2[...] = jnp.zeros_like(acc2)

        acc1[...] += jnp.sum(t, axis=0, keepdims=True)
        acc2[...] += jnp.sum(t * t, axis=0, keepdims=True)

        @pl.when(i == GRID - 1)
        def _():
            mean = acc1[...] / N
            var = acc2[...] / N - mean * mean
            scale = g_ref[...] * lax.rsqrt(var + 1e-5)
            st[0:1, :] = scale
            st[1:2, :] = bt_ref[...] - mean * scale

    def body2(t_ref, st_ref, wz_ref, bz_ref, z_ref):
        a = jnp.maximum(t_ref[...] * st_ref[0:1, :] + st_ref[1:2, :], 0.0)
        z = jnp.dot(a, wz_ref[...], preferred_element_type=jnp.float32)
        z_ref[...] = z + bz_ref[...]

    t_en, st = pl.pallas_call(
        body,
        grid=(GRID,),
        in_specs=[
            pl.BlockSpec((BM, D), lambda i: (i, 0)),
            pl.BlockSpec((D, H), lambda i: (0, 0)),
            pl.BlockSpec((1, H), lambda i: (0, 0)),
            pl.BlockSpec((1, H), lambda i: (0, 0)),
            pl.BlockSpec((1, H), lambda i: (0, 0)),
        ],
        out_specs=[
            pl.BlockSpec((BM, H), lambda i: (i, 0)),
            pl.BlockSpec((2, H), lambda i: (0, 0)),
        ],
        out_shape=[
            jax.ShapeDtypeStruct((N, H), jnp.float32),
            jax.ShapeDtypeStruct((2, H), jnp.float32),
        ],
        scratch_shapes=[
            pltpu.VMEM((1, H), jnp.float32),
            pltpu.VMEM((1, H), jnp.float32),
        ],
    )(x, wet, be, g, beta)
    return pl.pallas_call(
        body2,
        grid=(GRID,),
        in_specs=[
            pl.BlockSpec((BM, H), lambda i: (i, 0)),
            pl.BlockSpec((2, H), lambda i: (0, 0)),
            pl.BlockSpec((H, Z), lambda i: (0, 0)),
            pl.BlockSpec((1, Z), lambda i: (0, 0)),
        ],
        out_specs=pl.BlockSpec((BM, Z), lambda i: (i, 0)),
        out_shape=jax.ShapeDtypeStruct((N, Z), jnp.float32),
    )(t_en, st, wezt, bez)


def _tc_stage2(x, agg0, fc0at, fc0bt, fb0, w1t, b1):
    def body(x_ref, a_ref, wa_ref, wb_ref, fb_ref,
             w1_ref, b1_ref, o_ref, h1_ref):
        o = jnp.dot(x_ref[...], wa_ref[...], preferred_element_type=jnp.float32)
        o += jnp.dot(a_ref[...], wb_ref[...], preferred_element_type=jnp.float32)
        o = jnp.maximum(o + fb_ref[...], 0.0)
        o_ref[...] = o
        h1 = jnp.dot(o, w1_ref[...], preferred_element_type=jnp.float32)
        h1_ref[...] = jnp.maximum(h1 + b1_ref[...], 0.0)

    return pl.pallas_call(
        body,
        grid=(GRID,),
        in_specs=[
            pl.BlockSpec((BM, D), lambda i: (i, 0)),
            pl.BlockSpec((BM, D), lambda i: (i, 0)),
            pl.BlockSpec((D, H), lambda i: (0, 0)),
            pl.BlockSpec((D, H), lambda i: (0, 0)),
            pl.BlockSpec((1, H), lambda i: (0, 0)),
            pl.BlockSpec((H, H), lambda i: (0, 0)),
            pl.BlockSpec((1, H), lambda i: (0, 0)),
        ],
        out_specs=[
            pl.BlockSpec((BM, H), lambda i: (i, 0)),
            pl.BlockSpec((BM, H), lambda i: (i, 0)),
        ],
        out_shape=[
            jax.ShapeDtypeStruct((N, H), jnp.float32),
            jax.ShapeDtypeStruct((N, H), jnp.float32),
        ],
    )(x, agg0, fc0at, fc0bt, fb0, w1t, b1)


def _tc_stage3(out1, agg1, ae_z, fc1at, fc1bt, fb1, wdht, bdh, g, beta):
    def body(o_ref, a_ref, ez_ref, wa_ref, wb_ref, fb_ref, wd_ref, bd_ref,
             g_ref, bt_ref, gs_ref, z_ref, td_ref, st_ref, acc1, acc2):
        i = pl.program_id(0)
        gs = jnp.dot(o_ref[...], wa_ref[...], preferred_element_type=jnp.float32)
        gs += jnp.dot(a_ref[...], wb_ref[...], preferred_element_type=jnp.float32)
        gs = gs + fb_ref[...]
        gs_ref[...] = gs
        z = 0.5 * ez_ref[...] + 0.5 * gs
        z_ref[...] = z
        t = jnp.dot(z, wd_ref[...], preferred_element_type=jnp.float32)
        t = t + bd_ref[...]
        td_ref[...] = t

        @pl.when(i == 0)
        def _():
            acc1[...] = jnp.zeros_like(acc1)
            acc2[...] = jnp.zeros_like(acc2)

        acc1[...] += jnp.sum(t, axis=0, keepdims=True)
        acc2[...] += jnp.sum(t * t, axis=0, keepdims=True)

        @pl.when(i == GRID - 1)
        def _():
            mean = acc1[...] / N
            var = acc2[...] / N - mean * mean
            scale = g_ref[...] * lax.rsqrt(var + 1e-5)
            st_ref[0:1, :] = scale
            st_ref[1:2, :] = bt_ref[...] - mean * scale

    return pl.pallas_call(
        body,
        grid=(GRID,),
        in_specs=[
            pl.BlockSpec((BM, H), lambda i: (i, 0)),
            pl.BlockSpec((BM, H), lambda i: (i, 0)),
            pl.BlockSpec((BM, Z), lambda i: (i, 0)),
            pl.BlockSpec((H, Z), lambda i: (0, 0)),
            pl.BlockSpec((H, Z), lambda i: (0, 0)),
            pl.BlockSpec((1, Z), lambda i: (0, 0)),
            pl.BlockSpec((Z, H), lambda i: (0, 0)),
            pl.BlockSpec((1, H), lambda i: (0, 0)),
            pl.BlockSpec((1, H), lambda i: (0, 0)),
            pl.BlockSpec((1, H), lambda i: (0, 0)),
        ],
        out_specs=[
            pl.BlockSpec((BM, Z), lambda i: (i, 0)),
            pl.BlockSpec((BM, Z), lambda i: (i, 0)),
            pl.BlockSpec((BM, H), lambda i: (i, 0)),
            pl.BlockSpec((2, H), lambda i: (0, 0)),
        ],
        out_shape=[
            jax.ShapeDtypeStruct((N, Z), jnp.float32),
            jax.ShapeDtypeStruct((N, Z), jnp.float32),
            jax.ShapeDtypeStruct((N, H), jnp.float32),
            jax.ShapeDtypeStruct((2, H), jnp.float32),
        ],
        scratch_shapes=[
            pltpu.VMEM((1, H), jnp.float32),
            pltpu.VMEM((1, H), jnp.float32),
        ],
    )(out1, agg1, ae_z, fc1at, fc1bt, fb1, wdht, bdh, g, beta)


def _tc_stage4(t_de, st, wdxt, bdx):
    def body(t_ref, st_ref, w_ref, b_ref, xb_ref):
        dd = jnp.maximum(t_ref[...] * st_ref[0:1, :] + st_ref[1:2, :], 0.0)
        xb = jnp.dot(dd, w_ref[...], preferred_element_type=jnp.float32)
        xb_ref[...] = xb + b_ref[...]

    return pl.pallas_call(
        body,
        grid=(GRID,),
        in_specs=[
            pl.BlockSpec((BM, H), lambda i: (i, 0)),
            pl.BlockSpec((2, H), lambda i: (0, 0)),
            pl.BlockSpec((H, D), lambda i: (0, 0)),
            pl.BlockSpec((1, D), lambda i: (0, 0)),
        ],
        out_specs=pl.BlockSpec((BM, D), lambda i: (i, 0)),
        out_shape=jax.ShapeDtypeStruct((N, D), jnp.float32),
    )(t_de, st, wdxt, bdx)


def kernel(x, neigh_idx0, neigh_idx1, agg0_W, agg0_b, agg1_W, agg1_b,
           fc0_W, fc0_b, fc1_W, fc1_b, en_h_W, en_h_b, en_g, en_beta,
           en_z_W, en_z_b, de_h_W, de_h_b, de_g, de_beta, de_x_W, de_x_b):
    r = lambda v: v.reshape(1, -1)
    idx_0 = _prep_idx(neigh_idx0)
    idx_1 = _prep_idx(neigh_idx1)

    h0_all = _tc_stage1(x, agg0_W.T, r(agg0_b))
    agg0 = _gather_max(h0_all, idx_0)
    ae_z = _tc_ae(x, en_h_W.T, r(en_h_b), r(en_g), r(en_beta),
                  en_z_W.T, r(en_z_b))
    out1, h1_all = _tc_stage2(
        x, agg0, fc0_W[:, :D].T, fc0_W[:, D:].T, r(fc0_b),
        agg1_W.T, r(agg1_b))
    agg1 = _gather_max(h1_all, idx_1)
    gs_z, z, t_de, de_st = _tc_stage3(
        out1, agg1, ae_z, fc1_W[:, :H].T, fc1_W[:, H:].T, r(fc1_b),
        de_h_W.T, r(de_h_b), r(de_g), r(de_beta))
    x_bar = _tc_stage4(t_de, de_st, de_x_W.T, r(de_x_b))
    return (ae_z, gs_z, z, x_bar, x)
